# Initial kernel scaffold; baseline (speedup 1.0000x reference)
import jax
import jax.numpy as jnp
from jax import lax
from jax.experimental import pallas as pl
from jax.experimental.pallas import tpu as pltpu
from jax.experimental.pallas import tpu_sc as plsc

N = 100000
E = 3200000
NUM_GRAPHS = 1000
HID = 64
IN_FEAT = 14

NC = 2
NS = 16
L = 16

BLK = 512
NB = 392
EPT = NB * BLK
E_PAD = EPT * NS
CHUNK = 25088
NCHUNK = 4
N_PAD = NCHUNK * CHUNK
DUMMY = 256
ACC_ROWS = CHUNK + DUMMY
ZROWS = 132
WOUT = CHUNK // NS

NB_D = 196
EPT_D = NB_D * BLK
ACC_D = N_PAD + DUMMY
ZROWS_D = 524
WOUT_D = N_PAD // NS

PBLK = 625
PNB = 5

_mesh = plsc.VectorSubcoreMesh(core_axis_name="c", subcore_axis_name="s")


def _zero_vmem(ref, rows, cols):
    @pl.loop(0, rows)
    def _(r):
        for k in range(cols // L):
            ref[r, pl.ds(k * L, L)] = jnp.zeros((L,), jnp.float32)


def _deg_body(dstp, dout, acc, zbuf, ones, di0, di1, dl0, dl1,
              s_di0, s_di1, s_sc0, s_sc1):
    cid = lax.axis_index("c")
    sid = lax.axis_index("s")
    wid = cid * NS + sid
    ebase = wid * EPT_D
    di = (di0, di1)
    dl = (dl0, dl1)
    s_di = (s_di0, s_di1)
    s_sc = (s_sc0, s_sc1)

    _zero_vmem(zbuf, ZROWS_D, L)

    @pl.loop(0, BLK)
    def _(r):
        ones[r, pl.ds(0, L)] = jnp.ones((L,), jnp.float32)

    for j in range(12):
        pltpu.sync_copy(
            zbuf, acc.at[pl.ds(sid * (12 * ZROWS_D) + j * ZROWS_D, ZROWS_D)])
    plsc.subcore_barrier()

    for s in range(2):
        pltpu.async_copy(dstp.at[pl.ds(ebase + s * BLK, BLK)], di[s], s_di[s])

    @pl.loop(0, NB_D // 2)
    def _(io):
        for s in range(2):
            i = io * 2 + s
            pltpu.make_async_copy(dstp.at[pl.ds(ebase, BLK)], di[s],
                                  s_di[s]).wait()

            @pl.when(i >= 2)
            def _():
                pltpu.make_async_copy(ones, acc.at[dl[s]], s_sc[s]).wait()

            @pl.loop(0, BLK // L)
            def _(j):
                v = di[s][pl.ds(j * L, L)]
                ok = v < N_PAD
                dl[s][pl.ds(j * L, L)] = jnp.where(
                    ok, v, N_PAD + (v & (DUMMY - 1)))

            @pl.when(i + 2 < NB_D)
            def _():
                pltpu.async_copy(dstp.at[pl.ds(ebase + (i + 2) * BLK, BLK)],
                                 di[s], s_di[s])

            pltpu.async_copy(ones, acc.at[dl[s]], s_sc[s], add=True)

    for s in range(2):
        pltpu.make_async_copy(ones, acc.at[dl[s]], s_sc[s]).wait()
    plsc.subcore_barrier()

    pltpu.sync_copy(acc.at[pl.ds(sid * WOUT_D, WOUT_D)],
                    dout.at[cid, pl.ds(sid * WOUT_D, WOUT_D)])


def _deg_call(dstp):
    return pl.kernel(
        _deg_body,
        out_type=jax.ShapeDtypeStruct((NC, N_PAD, L), jnp.float32),
        mesh=_mesh,
        scratch_types=[
            pltpu.VMEM_SHARED((ACC_D, L), jnp.float32),
            pltpu.VMEM((ZROWS_D, L), jnp.float32),
            pltpu.VMEM((BLK, L), jnp.float32),
            pltpu.VMEM((BLK,), jnp.int32),
            pltpu.VMEM((BLK,), jnp.int32),
            pltpu.VMEM((BLK,), jnp.int32),
            pltpu.VMEM((BLK,), jnp.int32),
            pltpu.SemaphoreType.DMA,
            pltpu.SemaphoreType.DMA,
            pltpu.SemaphoreType.DMA,
            pltpu.SemaphoreType.DMA,
        ],
    )(dstp)


def _agg_body(zp, srcp, dstp, sout, acc, zbuf,
              si0, si1, di0, di1, dl0, dl1, rw0, rw1,
              s_si0, s_si1, s_di0, s_di1, s_g0, s_g1, s_sc0, s_sc1):
    cid = lax.axis_index("c")
    sid = lax.axis_index("s")
    ebase = sid * EPT
    si = (si0, si1)
    di = (di0, di1)
    dl = (dl0, dl1)
    rw = (rw0, rw1)
    s_si = (s_si0, s_si1)
    s_di = (s_di0, s_di1)
    s_g = (s_g0, s_g1)
    s_sc = (s_sc0, s_sc1)

    _zero_vmem(zbuf, ZROWS, HID)

    for k_local in range(2):
        c0 = (cid * 2 + k_local) * CHUNK

        for j in range(12):
            pltpu.sync_copy(
                zbuf, acc.at[pl.ds(sid * (12 * ZROWS) + j * ZROWS, ZROWS)])
        plsc.subcore_barrier()

        for s in range(2):
            pltpu.async_copy(srcp.at[pl.ds(ebase + s * BLK, BLK)], si[s],
                             s_si[s])
            pltpu.async_copy(dstp.at[pl.ds(ebase + s * BLK, BLK)], di[s],
                             s_di[s])

        @pl.loop(0, NB // 2)
        def _(io):
            for s in range(2):
                i = io * 2 + s
                pltpu.make_async_copy(srcp.at[pl.ds(ebase, BLK)], si[s],
                                      s_si[s]).wait()
                pltpu.make_async_copy(dstp.at[pl.ds(ebase, BLK)], di[s],
                                      s_di[s]).wait()

                @pl.when(i >= 2)
                def _():
                    pltpu.make_async_copy(rw[s], acc.at[dl[s]], s_sc[s]).wait()

                pltpu.async_copy(zp.at[si[s]], rw[s], s_g[s])

                @pl.loop(0, BLK // L)
                def _(j):
                    v = di[s][pl.ds(j * L, L)]
                    loc = v - c0
                    ok = (loc >= 0) & (loc < CHUNK)
                    dl[s][pl.ds(j * L, L)] = jnp.where(
                        ok, loc, CHUNK + (v & (DUMMY - 1)))

                pltpu.make_async_copy(zp.at[si[s]], rw[s], s_g[s]).wait()

                @pl.when(i + 2 < NB)
                def _():
                    pltpu.async_copy(
                        srcp.at[pl.ds(ebase + (i + 2) * BLK, BLK)], si[s],
                        s_si[s])
                    pltpu.async_copy(
                        dstp.at[pl.ds(ebase + (i + 2) * BLK, BLK)], di[s],
                        s_di[s])

                pltpu.async_copy(rw[s], acc.at[dl[s]], s_sc[s], add=True)

        for s in range(2):
            pltpu.make_async_copy(rw[s], acc.at[dl[s]], s_sc[s]).wait()
        plsc.subcore_barrier()

        pltpu.sync_copy(acc.at[pl.ds(sid * WOUT, WOUT)],
                        sout.at[pl.ds(c0 + sid * WOUT, WOUT)])
        plsc.subcore_barrier()


def _agg_call(zp, srcp, dstp):
    return pl.kernel(
        _agg_body,
        out_type=jax.ShapeDtypeStruct((N_PAD, HID), jnp.float32),
        mesh=_mesh,
        scratch_types=[
            pltpu.VMEM_SHARED((ACC_ROWS, HID), jnp.float32),
            pltpu.VMEM((ZROWS, HID), jnp.float32),
            pltpu.VMEM((BLK,), jnp.int32),
            pltpu.VMEM((BLK,), jnp.int32),
            pltpu.VMEM((BLK,), jnp.int32),
            pltpu.VMEM((BLK,), jnp.int32),
            pltpu.VMEM((BLK,), jnp.int32),
            pltpu.VMEM((BLK,), jnp.int32),
            pltpu.VMEM((BLK, HID), jnp.float32),
            pltpu.VMEM((BLK, HID), jnp.float32),
            pltpu.SemaphoreType.DMA,
            pltpu.SemaphoreType.DMA,
            pltpu.SemaphoreType.DMA,
            pltpu.SemaphoreType.DMA,
            pltpu.SemaphoreType.DMA,
            pltpu.SemaphoreType.DMA,
            pltpu.SemaphoreType.DMA,
            pltpu.SemaphoreType.DMA,
        ],
    )(zp, srcp, dstp)


def _pool_body(h, bat, gout, acc, zbuf, bi0, bi1, rw0, rw1,
               s_b0, s_b1, s_r0, s_r1, s_sc0, s_sc1):
    cid = lax.axis_index("c")
    sid = lax.axis_index("s")
    wid = cid * NS + sid
    nbase = wid * (PNB * PBLK)
    bi = (bi0, bi1)
    rw = (rw0, rw1)
    s_b = (s_b0, s_b1)
    s_r = (s_r0, s_r1)
    s_sc = (s_sc0, s_sc1)

    _zero_vmem(zbuf, 125, HID)

    @pl.when(sid < 8)
    def _():
        pltpu.sync_copy(zbuf, acc.at[pl.ds(sid * 125, 125)])
    plsc.subcore_barrier()

    for s in range(2):
        pltpu.async_copy(bat.at[pl.ds(nbase + s * PBLK, PBLK)], bi[s], s_b[s])
        pltpu.async_copy(h.at[pl.ds(nbase + s * PBLK, PBLK)], rw[s], s_r[s])

    for i in range(PNB):
        s = i % 2
        if i >= 2:
            pltpu.make_async_copy(rw[s], acc.at[bi[s]], s_sc[s]).wait()
            pltpu.async_copy(bat.at[pl.ds(nbase + i * PBLK, PBLK)], bi[s],
                             s_b[s])
            pltpu.async_copy(h.at[pl.ds(nbase + i * PBLK, PBLK)], rw[s],
                             s_r[s])
        pltpu.make_async_copy(bat.at[pl.ds(nbase, PBLK)], bi[s], s_b[s]).wait()
        pltpu.make_async_copy(h.at[pl.ds(nbase, PBLK)], rw[s], s_r[s]).wait()
        pltpu.async_copy(rw[s], acc.at[bi[s]], s_sc[s], add=True)

    for s in range(2):
        pltpu.make_async_copy(rw[s], acc.at[bi[s]], s_sc[s]).wait()
    plsc.subcore_barrier()

    @pl.when(sid == 0)
    def _():
        pltpu.sync_copy(acc, gout.at[cid])


def _pool_call(h, bat):
    return pl.kernel(
        _pool_body,
        out_type=jax.ShapeDtypeStruct((NC, NUM_GRAPHS, HID), jnp.float32),
        mesh=_mesh,
        scratch_types=[
            pltpu.VMEM_SHARED((NUM_GRAPHS, HID), jnp.float32),
            pltpu.VMEM((125, HID), jnp.float32),
            pltpu.VMEM((PBLK,), jnp.int32),
            pltpu.VMEM((PBLK,), jnp.int32),
            pltpu.VMEM((PBLK, HID), jnp.float32),
            pltpu.VMEM((PBLK, HID), jnp.float32),
            pltpu.SemaphoreType.DMA,
            pltpu.SemaphoreType.DMA,
            pltpu.SemaphoreType.DMA,
            pltpu.SemaphoreType.DMA,
            pltpu.SemaphoreType.DMA,
            pltpu.SemaphoreType.DMA,
        ],
    )(h, bat)


MB = 1000
MG = N // MB


def _mm_body(h_ref, w_ref, dv_ref, o_ref):
    o_ref[...] = (jnp.dot(h_ref[...], w_ref[...],
                          preferred_element_type=jnp.float32) * dv_ref[...])


def _matmul_scale(h, w, dv):
    f = h.shape[1]
    return pl.pallas_call(
        _mm_body,
        grid=(MG,),
        in_specs=[
            pl.BlockSpec((MB, f), lambda i: (i, 0)),
            pl.BlockSpec((f, HID), lambda i: (0, 0)),
            pl.BlockSpec((MB, 1), lambda i: (i, 0)),
        ],
        out_specs=pl.BlockSpec((MB, HID), lambda i: (i, 0)),
        out_shape=jax.ShapeDtypeStruct((N, HID), jnp.float32),
    )(h, w, dv)


def _stats_body(s_ref, zp_ref, dv_ref, b_ref, o_ref):
    pre = (s_ref[...] + zp_ref[...]) * dv_ref[...] + b_ref[...]
    part = jnp.concatenate([jnp.sum(pre, axis=0, keepdims=True),
                            jnp.sum(pre * pre, axis=0, keepdims=True)], axis=0)

    @pl.when(pl.program_id(0) == 0)
    def _():
        o_ref[...] = jnp.zeros_like(o_ref)

    o_ref[...] += part


def _stats(s_pad, zp, dv, b):
    return pl.pallas_call(
        _stats_body,
        grid=(MG,),
        in_specs=[
            pl.BlockSpec((MB, HID), lambda i: (i, 0)),
            pl.BlockSpec((MB, HID), lambda i: (i, 0)),
            pl.BlockSpec((MB, 1), lambda i: (i, 0)),
            pl.BlockSpec((1, HID), lambda i: (0, 0)),
        ],
        out_specs=pl.BlockSpec((2, HID), lambda i: (0, 0)),
        out_shape=jax.ShapeDtypeStruct((2, HID), jnp.float32),
    )(s_pad, zp, dv, b)


def _apply_body(s_ref, zp_ref, dv_ref, ab_ref, o_ref):
    pre = (s_ref[...] + zp_ref[...]) * dv_ref[...]
    o_ref[...] = jnp.maximum(pre * ab_ref[0:1, :] + ab_ref[1:2, :], 0.0)


def _bn_apply(s_pad, zp, dv, ab):
    return pl.pallas_call(
        _apply_body,
        grid=(MG,),
        in_specs=[
            pl.BlockSpec((MB, HID), lambda i: (i, 0)),
            pl.BlockSpec((MB, HID), lambda i: (i, 0)),
            pl.BlockSpec((MB, 1), lambda i: (i, 0)),
            pl.BlockSpec((2, HID), lambda i: (0, 0)),
        ],
        out_specs=pl.BlockSpec((MB, HID), lambda i: (i, 0)),
        out_shape=jax.ShapeDtypeStruct((N, HID), jnp.float32),
    )(s_pad, zp, dv, ab)


DINV_ROWS = N_PAD * L // 128
DINV_BLK = 784


def _dinv_body(a_ref, o_ref):
    o_ref[...] = lax.rsqrt(1.0 + a_ref[0] + a_ref[1])


def _dinv(dpart):
    return pl.pallas_call(
        _dinv_body,
        grid=(DINV_ROWS // DINV_BLK,),
        in_specs=[pl.BlockSpec((NC, DINV_BLK, 128), lambda i: (0, i, 0))],
        out_specs=pl.BlockSpec((DINV_BLK, 128), lambda i: (i, 0)),
        out_shape=jax.ShapeDtypeStruct((DINV_ROWS, 128), jnp.float32),
    )(dpart)


def _final_body(g_ref, w_ref, b_ref, ge_ref, o_ref):
    ge = g_ref[0] + g_ref[1]
    ge_ref[...] = ge
    o_ref[...] = jnp.dot(ge, w_ref[...],
                         preferred_element_type=jnp.float32) + b_ref[...]


def _final(g, wfc, bfc):
    return pl.pallas_call(
        _final_body,
        out_shape=[
            jax.ShapeDtypeStruct((NUM_GRAPHS, HID), jnp.float32),
            jax.ShapeDtypeStruct((NUM_GRAPHS, 2), jnp.float32),
        ],
    )(g, wfc, bfc.reshape(1, 2))


def _layer(h, w, b, g, be, dv, srcp, dstp):
    zp = _matmul_scale(h, w, dv)
    s_pad = _agg_call(zp, srcp, dstp)
    sums = _stats(s_pad, zp, dv, b.reshape(1, HID))
    mu = sums[0] / N
    var = sums[1] / N - mu * mu
    scale = g * lax.rsqrt(var + 1e-5)
    cvec = (b - mu) * scale + be
    ab = jnp.stack([scale, cvec])
    return _bn_apply(s_pad, zp, dv, ab)


def kernel(x, edge_index, batch, W1, b1, g1, be1, W2, b2, g2, be2,
           W3, b3, g3, be3, Wfc, bfc):
    src = edge_index[0]
    dst = edge_index[1]
    npad = E_PAD - E
    srcp = jnp.concatenate(
        [src, (jnp.arange(npad, dtype=jnp.int32) * 9973) % N])
    dstp = jnp.concatenate(
        [dst, (1 << 29) + (jnp.arange(npad, dtype=jnp.int32) & 255)])

    dpart = _deg_call(dstp)
    dinv_w = _dinv(dpart.reshape(NC, DINV_ROWS, 128))
    dv = dinv_w.reshape(N_PAD, L)[:N, :1]

    h1 = _layer(x, W1, b1, g1, be1, dv, srcp, dstp)
    h2 = _layer(h1, W2, b2, g2, be2, dv, srcp, dstp)
    h3 = _layer(h2, W3, b3, g3, be3, dv, srcp, dstp)

    gpart = _pool_call(h3, batch)
    graph_emb, out = _final(gpart, Wfc, bfc)
    return (out, h3, graph_emb)


# speedup vs baseline: 9.3474x; 9.3474x over previous
import jax
import jax.numpy as jnp
from jax import lax
from jax.experimental import pallas as pl
from jax.experimental.pallas import tpu as pltpu
from jax.experimental.pallas import tpu_sc as plsc

N = 100000
E = 3200000
NUM_GRAPHS = 1000
HID = 64
IN_FEAT = 14

NC = 2
NS = 16
L = 16

BLK = 128
NB = 1564
EPT = NB * BLK
E_PAD = EPT * NS
CHUNK = 25088
NCHUNK = 4
N_PAD = NCHUNK * CHUNK
DUMMY = 64
ACC_ROWS = CHUNK + DUMMY
ZROWS = 131
WOUT = CHUNK // NS

EPW = E_PAD // (NC * NS)
NB_P = EPW // BLK
EPT_PART = EPW + BLK
SCAP = 256
PART_TOT = NCHUNK * NC * NS * EPT_PART

NB_D = 782
EPT_D = NB_D * BLK
ACC_D = N_PAD + DUMMY
ZROWS_D = 523
WOUT_D = N_PAD // NS

PBLK = 624
PNB = 5
PTAIL = N - NC * NS * PNB * PBLK
PTBASE = N - PTAIL

def _mesh():
    return plsc.VectorSubcoreMesh(core_axis_name="c", subcore_axis_name="s",
                                  num_cores=NC, num_subcores=NS)


def _zero_vmem(ref, rows, cols):
    @pl.loop(0, rows)
    def _(r):
        for k in range(cols // L):
            ref[r, pl.ds(k * L, L)] = jnp.zeros((L,), jnp.float32)


def _deg_body(dstp, dout, acc, zbuf, ones, di0, di1, dl0, dl1,
              s_di0, s_di1, s_sc0, s_sc1):
    cid = lax.axis_index("c")
    sid = lax.axis_index("s")
    wid = cid * NS + sid
    ebase = wid * EPT_D
    di = (di0, di1)
    dl = (dl0, dl1)
    s_di = (s_di0, s_di1)
    s_sc = (s_sc0, s_sc1)

    _zero_vmem(zbuf, ZROWS_D, L)

    @pl.loop(0, BLK)
    def _(r):
        ones[r, pl.ds(0, L)] = jnp.ones((L,), jnp.float32)

    for j in range(12):
        pltpu.sync_copy(
            zbuf, acc.at[pl.ds(sid * (12 * ZROWS_D) + j * ZROWS_D, ZROWS_D)])
    plsc.subcore_barrier()

    for s in range(2):
        pltpu.async_copy(dstp.at[pl.ds(ebase + s * BLK, BLK)], di[s], s_di[s])

    @pl.loop(0, NB_D // 2)
    def _(io):
        for s in range(2):
            i = io * 2 + s
            pltpu.make_async_copy(dstp.at[pl.ds(ebase, BLK)], di[s],
                                  s_di[s]).wait()

            @pl.when(i >= 2)
            def _():
                pltpu.make_async_copy(ones, acc.at[dl[s]], s_sc[s]).wait()

            @pl.loop(0, BLK // L)
            def _(j):
                v = di[s][pl.ds(j * L, L)]
                ok = v < N_PAD
                dl[s][pl.ds(j * L, L)] = jnp.where(
                    ok, v, N_PAD + (v & (DUMMY - 1)))

            @pl.when(i + 2 < NB_D)
            def _():
                pltpu.async_copy(dstp.at[pl.ds(ebase + (i + 2) * BLK, BLK)],
                                 di[s], s_di[s])

            pltpu.async_copy(ones, acc.at[dl[s]], s_sc[s], add=True)

    for s in range(2):
        pltpu.make_async_copy(ones, acc.at[dl[s]], s_sc[s]).wait()
    plsc.subcore_barrier()

    pltpu.sync_copy(acc.at[pl.ds(sid * WOUT_D, WOUT_D)],
                    dout.at[cid, pl.ds(sid * WOUT_D, WOUT_D)])


def _deg_call(dstp):
    return pl.kernel(
        _deg_body,
        out_type=jax.ShapeDtypeStruct((NC, N_PAD, L), jnp.float32),
        mesh=_mesh(),
        compiler_params=pltpu.CompilerParams(use_tc_tiling_on_sc=False),
        scratch_types=[
            pltpu.VMEM_SHARED((ACC_D, L), jnp.float32),
            pltpu.VMEM((ZROWS_D, L), jnp.float32),
            pltpu.VMEM((BLK, L), jnp.float32),
            pltpu.VMEM((BLK,), jnp.int32),
            pltpu.VMEM((BLK,), jnp.int32),
            pltpu.VMEM((BLK,), jnp.int32),
            pltpu.VMEM((BLK,), jnp.int32),
            pltpu.SemaphoreType.DMA,
            pltpu.SemaphoreType.DMA,
            pltpu.SemaphoreType.DMA,
            pltpu.SemaphoreType.DMA,
        ],
    )(dstp)


def _agg_body(zp, srcp, dstp, sout, acc, zbuf,
              si0, si1, di0, di1, dl0, dl1, rw0, rw1,
              s_si0, s_si1, s_di0, s_di1, s_g0, s_g1, s_sc0, s_sc1):
    cid = lax.axis_index("c")
    sid = lax.axis_index("s")
    ebase = sid * EPT
    si = (si0, si1)
    di = (di0, di1)
    dl = (dl0, dl1)
    rw = (rw0, rw1)
    s_si = (s_si0, s_si1)
    s_di = (s_di0, s_di1)
    s_g = (s_g0, s_g1)
    s_sc = (s_sc0, s_sc1)

    _zero_vmem(zbuf, ZROWS, HID)

    for k_local in range(2):
        c0 = (cid * 2 + k_local) * CHUNK

        for j in range(12):
            pltpu.sync_copy(
                zbuf, acc.at[pl.ds(sid * (12 * ZROWS) + j * ZROWS, ZROWS)])
        plsc.subcore_barrier()

        for s in range(2):
            pltpu.async_copy(srcp.at[pl.ds(ebase + s * BLK, BLK)], si[s],
                             s_si[s])
            pltpu.async_copy(dstp.at[pl.ds(ebase + s * BLK, BLK)], di[s],
                             s_di[s])

        @pl.loop(0, NB // 2)
        def _(io):
            for s in range(2):
                i = io * 2 + s
                pltpu.make_async_copy(srcp.at[pl.ds(ebase, BLK)], si[s],
                                      s_si[s]).wait()
                pltpu.make_async_copy(dstp.at[pl.ds(ebase, BLK)], di[s],
                                      s_di[s]).wait()

                @pl.when(i >= 2)
                def _():
                    pltpu.make_async_copy(rw[s], acc.at[dl[s]], s_sc[s]).wait()

                pltpu.async_copy(zp.at[si[s]], rw[s], s_g[s])

                @pl.loop(0, BLK // L)
                def _(j):
                    v = di[s][pl.ds(j * L, L)]
                    loc = v - c0
                    ok = (loc >= 0) & (loc < CHUNK)
                    dl[s][pl.ds(j * L, L)] = jnp.where(
                        ok, loc, CHUNK + (v & (DUMMY - 1)))

                pltpu.make_async_copy(zp.at[si[s]], rw[s], s_g[s]).wait()

                @pl.when(i + 2 < NB)
                def _():
                    pltpu.async_copy(
                        srcp.at[pl.ds(ebase + (i + 2) * BLK, BLK)], si[s],
                        s_si[s])
                    pltpu.async_copy(
                        dstp.at[pl.ds(ebase + (i + 2) * BLK, BLK)], di[s],
                        s_di[s])

                pltpu.async_copy(rw[s], acc.at[dl[s]], s_sc[s], add=True)

        for s in range(2):
            pltpu.make_async_copy(rw[s], acc.at[dl[s]], s_sc[s]).wait()
        plsc.subcore_barrier()

        pltpu.sync_copy(acc.at[pl.ds(sid * WOUT, WOUT)],
                        sout.at[pl.ds(c0 + sid * WOUT, WOUT)])
        plsc.subcore_barrier()


def _agg_call(zp, srcp, dstp):
    return pl.kernel(
        _agg_body,
        out_type=jax.ShapeDtypeStruct((N_PAD, HID), jnp.float32),
        mesh=_mesh(),
        compiler_params=pltpu.CompilerParams(use_tc_tiling_on_sc=False),
        scratch_types=[
            pltpu.VMEM_SHARED((ACC_ROWS, HID), jnp.float32),
            pltpu.VMEM((ZROWS, HID), jnp.float32),
            pltpu.VMEM((BLK,), jnp.int32),
            pltpu.VMEM((BLK,), jnp.int32),
            pltpu.VMEM((BLK,), jnp.int32),
            pltpu.VMEM((BLK,), jnp.int32),
            pltpu.VMEM((BLK,), jnp.int32),
            pltpu.VMEM((BLK,), jnp.int32),
            pltpu.VMEM((BLK, HID), jnp.float32),
            pltpu.VMEM((BLK, HID), jnp.float32),
            pltpu.SemaphoreType.DMA,
            pltpu.SemaphoreType.DMA,
            pltpu.SemaphoreType.DMA,
            pltpu.SemaphoreType.DMA,
            pltpu.SemaphoreType.DMA,
            pltpu.SemaphoreType.DMA,
            pltpu.SemaphoreType.DMA,
            pltpu.SemaphoreType.DMA,
        ],
    )(zp, srcp, dstp)


def _pool_body(h, bat, gout, acc, zbuf, bi0, bi1, rw0, rw1, bit, rwt,
               s_b0, s_b1, s_r0, s_r1, s_sc0, s_sc1):
    cid = lax.axis_index("c")
    sid = lax.axis_index("s")
    wid = cid * NS + sid
    nbase = wid * (PNB * PBLK)
    bi = (bi0, bi1)
    rw = (rw0, rw1)
    s_b = (s_b0, s_b1)
    s_r = (s_r0, s_r1)
    s_sc = (s_sc0, s_sc1)

    _zero_vmem(zbuf, 125, HID)

    @pl.when(sid < 8)
    def _():
        pltpu.sync_copy(zbuf, acc.at[pl.ds(sid * 125, 125)])
    plsc.subcore_barrier()

    for s in range(2):
        pltpu.async_copy(bat.at[pl.ds(nbase + s * PBLK, PBLK)], bi[s], s_b[s])
        pltpu.async_copy(h.at[pl.ds(nbase + s * PBLK, PBLK)], rw[s], s_r[s])

    for i in range(PNB):
        s = i % 2
        if i >= 2:
            pltpu.make_async_copy(rw[s], acc.at[bi[s]], s_sc[s]).wait()
            pltpu.async_copy(bat.at[pl.ds(nbase + i * PBLK, PBLK)], bi[s],
                             s_b[s])
            pltpu.async_copy(h.at[pl.ds(nbase + i * PBLK, PBLK)], rw[s],
                             s_r[s])
        pltpu.make_async_copy(bat.at[pl.ds(nbase, PBLK)], bi[s], s_b[s]).wait()
        pltpu.make_async_copy(h.at[pl.ds(nbase, PBLK)], rw[s], s_r[s]).wait()
        pltpu.async_copy(rw[s], acc.at[bi[s]], s_sc[s], add=True)

    for s in range(2):
        pltpu.make_async_copy(rw[s], acc.at[bi[s]], s_sc[s]).wait()

    @pl.when(wid == 0)
    def _():
        pltpu.sync_copy(bat.at[pl.ds(PTBASE, PTAIL)], bit)
        pltpu.sync_copy(h.at[pl.ds(PTBASE, PTAIL)], rwt)
        pltpu.async_copy(rwt, acc.at[bit], s_sc0, add=True)
        pltpu.make_async_copy(rwt, acc.at[bit], s_sc0).wait()
    plsc.subcore_barrier()

    @pl.when(sid == 0)
    def _():
        pltpu.sync_copy(acc, gout.at[cid])


def _pool_call(h, bat):
    return pl.kernel(
        _pool_body,
        out_type=jax.ShapeDtypeStruct((NC, NUM_GRAPHS, HID), jnp.float32),
        mesh=_mesh(),
        compiler_params=pltpu.CompilerParams(use_tc_tiling_on_sc=False),
        scratch_types=[
            pltpu.VMEM_SHARED((NUM_GRAPHS, HID), jnp.float32),
            pltpu.VMEM((125, HID), jnp.float32),
            pltpu.VMEM((PBLK,), jnp.int32),
            pltpu.VMEM((PBLK,), jnp.int32),
            pltpu.VMEM((PBLK, HID), jnp.float32),
            pltpu.VMEM((PBLK, HID), jnp.float32),
            pltpu.VMEM((PTAIL,), jnp.int32),
            pltpu.VMEM((PTAIL, HID), jnp.float32),
            pltpu.SemaphoreType.DMA,
            pltpu.SemaphoreType.DMA,
            pltpu.SemaphoreType.DMA,
            pltpu.SemaphoreType.DMA,
            pltpu.SemaphoreType.DMA,
            pltpu.SemaphoreType.DMA,
        ],
    )(h, bat)


MB = 1000
MG = N // MB


def _mm_body(h_ref, w_ref, dv_ref, o_ref):
    o_ref[...] = (jnp.dot(h_ref[...], w_ref[...],
                          preferred_element_type=jnp.float32) * dv_ref[...])


def _matmul_scale(h, w, dv):
    f = h.shape[1]
    return pl.pallas_call(
        _mm_body,
        grid=(MG,),
        in_specs=[
            pl.BlockSpec((MB, f), lambda i: (i, 0)),
            pl.BlockSpec((f, HID), lambda i: (0, 0)),
            pl.BlockSpec((MB, 1), lambda i: (i, 0)),
        ],
        out_specs=pl.BlockSpec((MB, HID), lambda i: (i, 0)),
        out_shape=jax.ShapeDtypeStruct((N, HID), jnp.float32),
    )(h, w, dv)


def _stats_body(s_ref, zp_ref, dv_ref, b_ref, o_ref):
    pre = (s_ref[...] + zp_ref[...]) * dv_ref[...] + b_ref[...]
    part = jnp.concatenate([jnp.sum(pre, axis=0, keepdims=True),
                            jnp.sum(pre * pre, axis=0, keepdims=True)], axis=0)

    @pl.when(pl.program_id(0) == 0)
    def _():
        o_ref[...] = jnp.zeros_like(o_ref)

    o_ref[...] += part


def _stats(s_pad, zp, dv, b):
    return pl.pallas_call(
        _stats_body,
        grid=(MG,),
        in_specs=[
            pl.BlockSpec((MB, HID), lambda i: (i, 0)),
            pl.BlockSpec((MB, HID), lambda i: (i, 0)),
            pl.BlockSpec((MB, 1), lambda i: (i, 0)),
            pl.BlockSpec((1, HID), lambda i: (0, 0)),
        ],
        out_specs=pl.BlockSpec((2, HID), lambda i: (0, 0)),
        out_shape=jax.ShapeDtypeStruct((2, HID), jnp.float32),
    )(s_pad, zp, dv, b)


def _apply_body(s_ref, zp_ref, dv_ref, ab_ref, o_ref):
    pre = (s_ref[...] + zp_ref[...]) * dv_ref[...]
    o_ref[...] = jnp.maximum(pre * ab_ref[0:1, :] + ab_ref[1:2, :], 0.0)


def _bn_apply(s_pad, zp, dv, ab):
    return pl.pallas_call(
        _apply_body,
        grid=(MG,),
        in_specs=[
            pl.BlockSpec((MB, HID), lambda i: (i, 0)),
            pl.BlockSpec((MB, HID), lambda i: (i, 0)),
            pl.BlockSpec((MB, 1), lambda i: (i, 0)),
            pl.BlockSpec((2, HID), lambda i: (0, 0)),
        ],
        out_specs=pl.BlockSpec((MB, HID), lambda i: (i, 0)),
        out_shape=jax.ShapeDtypeStruct((N, HID), jnp.float32),
    )(s_pad, zp, dv, ab)


DINV_ROWS = N_PAD * L // 128
DINV_BLK = 784


def _dinv_body(a_ref, o_ref):
    o_ref[...] = lax.rsqrt(1.0 + a_ref[0] + a_ref[1])


def _dinv(dpart):
    return pl.pallas_call(
        _dinv_body,
        grid=(DINV_ROWS // DINV_BLK,),
        in_specs=[pl.BlockSpec((NC, DINV_BLK, 128), lambda i: (0, i, 0))],
        out_specs=pl.BlockSpec((DINV_BLK, 128), lambda i: (i, 0)),
        out_shape=jax.ShapeDtypeStruct((DINV_ROWS, 128), jnp.float32),
    )(dpart)


def _final_body(g_ref, w_ref, b_ref, ge_ref, o_ref):
    ge = g_ref[0] + g_ref[1]
    ge_ref[...] = ge
    o_ref[...] = jnp.dot(ge, w_ref[...],
                         preferred_element_type=jnp.float32) + b_ref[...]


def _final(g, wfc, bfc):
    return pl.pallas_call(
        _final_body,
        out_shape=[
            jax.ShapeDtypeStruct((NUM_GRAPHS, HID), jnp.float32),
            jax.ShapeDtypeStruct((NUM_GRAPHS, 2), jnp.float32),
        ],
    )(g, wfc, bfc.reshape(1, 2))


def _layer(h, w, b, g, be, dv, srcp, dstp):
    zp = _matmul_scale(h, w, dv)
    s_pad = _agg_call(zp, srcp, dstp)
    sums = _stats(s_pad, zp, dv, b.reshape(1, HID))
    mu = sums[0] / N
    var = sums[1] / N - mu * mu
    scale = g * lax.rsqrt(var + 1e-5)
    cvec = (b - mu) * scale + be
    ab = jnp.stack([scale, cvec])
    return _bn_apply(s_pad, zp, dv, ab)


def kernel(x, edge_index, batch, W1, b1, g1, be1, W2, b2, g2, be2,
           W3, b3, g3, be3, Wfc, bfc):
    src = edge_index[0]
    dst = edge_index[1]
    npad = E_PAD - E
    srcp = jnp.concatenate(
        [src, (jnp.arange(npad, dtype=jnp.int32) * 9973) % N])
    dstp = jnp.concatenate(
        [dst, (1 << 29) + (jnp.arange(npad, dtype=jnp.int32) & 255)])

    dpart = _deg_call(dstp)
    dinv_w = _dinv(dpart.reshape(NC, DINV_ROWS, 128))
    dv = dinv_w.reshape(N_PAD, L)[:N, :1]

    h1 = _layer(x, W1, b1, g1, be1, dv, srcp, dstp)
    h2 = _layer(h1, W2, b2, g2, be2, dv, srcp, dstp)
    h3 = _layer(h2, W3, b3, g3, be3, dv, srcp, dstp)

    gpart = _pool_call(h3, batch)
    graph_emb, out = _final(gpart, Wfc, bfc)
    return (out, h3, graph_emb)
